# baseline (speedup 1.0000x reference)
import functools

import jax
import jax.numpy as jnp
from jax import lax
from jax.experimental import pallas as pl
from jax.experimental.pallas import tpu as pltpu
from jax.experimental.pallas import tpu_sc as plsc

B = 16384
F = 26
V = 100000
D = 128
H = 1024
EPS = 1e-5

NC = 2
NS = 16
NW = NC * NS
SCH = 4
RCH = SCH * F
DV = D // 16

NSPLIT = 2
BSP = B // NSPLIT
BPW = BSP // NW
NCHUNK = BPW // SCH


def _sc_body(idx_hbm, table_hbm, mean_hbm, fm2_hbm, idx_v, rows_v, mean_v,
             fm2_v, sem0, sem1):
    wid = lax.axis_index("s") * NC + lax.axis_index("c")
    pltpu.sync_copy(idx_hbm.at[wid], idx_v)
    inv = jnp.float32(1.0 / F)
    sems = (sem0, sem1)

    def start(c, b):
        pltpu.make_async_copy(table_hbm.at[idx_v.at[c]], rows_v.at[b],
                              sems[b]).start()

    def wait(c, b):
        pltpu.make_async_copy(table_hbm.at[idx_v.at[c]], rows_v.at[b],
                              sems[b]).wait()

    def reduce(c, b):
        for j in range(SCH):
            ss = [jnp.zeros((16,), jnp.float32)] * DV
            qq = [jnp.zeros((16,), jnp.float32)] * DV
            for f in range(F):
                for d in range(DV):
                    v = rows_v[b, j * F + f, pl.ds(d * 16, 16)]
                    ss[d] = ss[d] + v
                    qq[d] = qq[d] + v * v
            s = c * SCH + j
            part = jnp.zeros((16,), jnp.float32)
            for d in range(DV):
                mean_v[s, pl.ds(d * 16, 16)] = ss[d] * inv
                part = part + (ss[d] * ss[d] - qq[d])
            fm2_v[c // 2, pl.ds((c % 2) * 64 + j * 16, 16)] = 0.5 * part

    start(0, 0)

    def pair(i, carry):
        c = i * 2
        start(c + 1, 1)
        wait(c, 0)
        reduce(c, 0)

        @pl.when(c + 2 < NCHUNK)
        def _():
            start(c + 2, 0)

        wait(c + 1, 1)
        reduce(c + 1, 1)
        return carry

    lax.fori_loop(0, NCHUNK // 2, pair, 0)
    pltpu.sync_copy(mean_v, mean_hbm.at[pl.ds(wid * BPW, BPW)])
    pltpu.sync_copy(fm2_v, fm2_hbm.at[pl.ds(wid * (BPW // 8), BPW // 8)])


_sc_reduce = functools.partial(
    pl.kernel,
    out_type=(jax.ShapeDtypeStruct((BSP, D), jnp.float32),
              jax.ShapeDtypeStruct((BSP // 8, D), jnp.float32)),
    mesh=plsc.VectorSubcoreMesh(core_axis_name="c", subcore_axis_name="s",
                                num_cores=NC, num_subcores=NS),
    scratch_types=[
        pltpu.VMEM((NCHUNK, RCH), jnp.int32),
        pltpu.VMEM((2, RCH, D), jnp.float32),
        pltpu.VMEM((BPW, D), jnp.float32),
        pltpu.VMEM((BPW // 8, D), jnp.float32),
        pltpu.SemaphoreType.DMA,
        pltpu.SemaphoreType.DMA,
    ],
)(_sc_body)


TB = 4096
NT = B // TB
NTH = BSP // TB


def _tc_raw_body(scal_ref, mean_ref, fm2_ref, si_ref, w1_ref, b1_ref, w2_ref,
                 fmr_ref, dpr_ref):
    m = mean_ref[...]
    h = lax.dot_general(m, w1_ref[...], (((1,), (1,)), ((), ())),
                        preferred_element_type=jnp.float32)
    h = jnp.maximum(h + b1_ref[...], 0.0)
    dp = lax.dot_general(h, w2_ref[...], (((1,), (1,)), ((), ())),
                         preferred_element_type=jnp.float32)
    fm1 = jnp.sum(si_ref[...].astype(jnp.float32), axis=1)
    fmr_ref[0, 0, :] = fm1 + jnp.sum(fm2_ref[...], axis=1)
    dpr_ref[0, 0, :] = dp[:, 0] + scal_ref[6]


def _tc_mlp2_body(scal_ref, mean_ref, fm2_ref, si_ref, w1_ref, b1_ref,
                  w2_ref, fmr0_ref, dpr0_ref, out_ref, fmr_s, dpr_s):
    c = pl.program_id(0)

    @pl.when(c < NTH)
    def _():
        m = mean_ref[...]
        h = lax.dot_general(m, w1_ref[...], (((1,), (1,)), ((), ())),
                            preferred_element_type=jnp.float32)
        h = jnp.maximum(h + b1_ref[...], 0.0)
        dp = lax.dot_general(h, w2_ref[...], (((1,), (1,)), ((), ())),
                             preferred_element_type=jnp.float32)
        fm1 = jnp.sum(si_ref[...].astype(jnp.float32), axis=1)
        fmr_s[c, :] = fm1 + jnp.sum(fm2_ref[...], axis=1)
        dpr_s[c, :] = dp[:, 0] + scal_ref[6]

    @pl.when(c == NTH)
    def _():
        fmr = jnp.concatenate([fmr0_ref[...].reshape(NTH, TB), fmr_s[...]],
                              axis=0)
        dpr = jnp.concatenate([dpr0_ref[...].reshape(NTH, TB), dpr_s[...]],
                              axis=0)
        fmean = jnp.mean(fmr)
        fvar = jnp.mean((fmr - fmean) ** 2)
        dmean = jnp.mean(dpr)
        dvar = jnp.mean((dpr - dmean) ** 2)
        fm_n = (scal_ref[0] * (fmr - fmean) * lax.rsqrt(fvar + EPS)
                + scal_ref[1])
        dp_n = (scal_ref[2] * (dpr - dmean) * lax.rsqrt(dvar + EPS)
                + scal_ref[3])
        out_ref[...] = jax.nn.sigmoid(scal_ref[4] * fm_n + scal_ref[5] * dp_n)


def _tc_mlp2(scal, mean_h, fm2_h, si_h, w1, b1r, w2, fmr0, dpr0):
    return pl.pallas_call(
        _tc_mlp2_body,
        grid=(NTH + 1,),
        in_specs=[
            pl.BlockSpec(memory_space=pltpu.SMEM),
            pl.BlockSpec((TB, D), lambda c: (jnp.minimum(c, NTH - 1), 0)),
            pl.BlockSpec((TB, 16), lambda c: (jnp.minimum(c, NTH - 1), 0)),
            pl.BlockSpec((TB, F), lambda c: (jnp.minimum(c, NTH - 1), 0)),
            pl.BlockSpec((H, D), lambda c: (0, 0)),
            pl.BlockSpec((1, H), lambda c: (0, 0)),
            pl.BlockSpec((1, H), lambda c: (0, 0)),
            pl.BlockSpec((NTH, 1, TB), lambda c: (0, 0, 0)),
            pl.BlockSpec((NTH, 1, TB), lambda c: (0, 0, 0)),
        ],
        out_specs=pl.BlockSpec((NT, TB), lambda c: (0, 0)),
        out_shape=jax.ShapeDtypeStruct((NT, TB), jnp.float32),
        scratch_shapes=[pltpu.VMEM((NTH, TB), jnp.float32),
                        pltpu.VMEM((NTH, TB), jnp.float32)],
    )(scal, mean_h, fm2_h, si_h, w1, b1r, w2, fmr0, dpr0)


def _tc_raw(scal, mean_h, fm2_h, si_h, w1, b1r, w2):
    return pl.pallas_call(
        _tc_raw_body,
        grid=(NTH,),
        in_specs=[
            pl.BlockSpec(memory_space=pltpu.SMEM),
            pl.BlockSpec((TB, D), lambda c: (c, 0)),
            pl.BlockSpec((TB, 16), lambda c: (c, 0)),
            pl.BlockSpec((TB, F), lambda c: (c, 0)),
            pl.BlockSpec((H, D), lambda c: (0, 0)),
            pl.BlockSpec((1, H), lambda c: (0, 0)),
            pl.BlockSpec((1, H), lambda c: (0, 0)),
        ],
        out_specs=[pl.BlockSpec((1, 1, TB), lambda c: (c, 0, 0)),
                   pl.BlockSpec((1, 1, TB), lambda c: (c, 0, 0))],
        out_shape=[jax.ShapeDtypeStruct((NTH, 1, TB), jnp.float32),
                   jax.ShapeDtypeStruct((NTH, 1, TB), jnp.float32)],
    )(scal, mean_h, fm2_h, si_h, w1, b1r, w2)


def kernel(sparse_inputs, emb_tables, w1, b1, w2, b2, fm_gamma, fm_beta,
           deep_gamma, deep_beta, combine_weight):
    si = sparse_inputs.astype(jnp.int32)
    flat = si + (jnp.arange(F, dtype=jnp.int32) * V)[None, :]
    idx = flat.reshape(NSPLIT, NW, NCHUNK, RCH)
    table = emb_tables.reshape(F * V, D)
    scal = jnp.concatenate([fm_gamma, fm_beta, deep_gamma, deep_beta,
                            combine_weight, b2]).astype(jnp.float32)
    b1r = b1.reshape(1, H)
    si_s = si.reshape(NSPLIT, BSP, F)

    means, fm2s = [], []
    for h in range(NSPLIT):
        mean_h, fm2_h = _sc_reduce(idx[h], table)
        means.append(mean_h)
        fm2s.append(fm2_h)

    fmr0, dpr0 = _tc_raw(scal, means[0], fm2s[0].reshape(BSP, 16),
                         si_s[0], w1, b1r, w2)
    out = _tc_mlp2(scal, means[1], fm2s[1].reshape(BSP, 16), si_s[1], w1,
                   b1r, w2, fmr0, dpr0)
    return out.reshape(B, 1)


# speedup vs baseline: 1.0721x; 1.0721x over previous
import functools

import jax
import jax.numpy as jnp
from jax import lax
from jax.experimental import pallas as pl
from jax.experimental.pallas import tpu as pltpu
from jax.experimental.pallas import tpu_sc as plsc

B = 16384
F = 26
V = 100000
D = 128
H = 1024
EPS = 1e-5

NC = 2
NS = 16
NW = NC * NS
SCH = 4
RCH = SCH * F
DV = D // 16

NSPLIT = 2
BSP = B // NSPLIT
BPW = BSP // NW
NCHUNK = BPW // SCH


def _sc_body(idx_hbm, table_hbm, mean_hbm, fm2_hbm, idx_v, rows_v, mean_v,
             fm2_v, sem0, sem1):
    wid = lax.axis_index("s") * NC + lax.axis_index("c")
    pltpu.sync_copy(idx_hbm.at[wid], idx_v)
    inv = jnp.float32(1.0 / F)
    sems = (sem0, sem1)

    def start(c, b):
        pltpu.make_async_copy(table_hbm.at[idx_v.at[c]], rows_v.at[b],
                              sems[b]).start()

    def wait(c, b):
        pltpu.make_async_copy(table_hbm.at[idx_v.at[c]], rows_v.at[b],
                              sems[b]).wait()

    def reduce(c, b):
        for j in range(SCH):
            ss = [jnp.zeros((16,), jnp.float32)] * DV
            qq = [jnp.zeros((16,), jnp.float32)] * DV
            for f in range(F):
                for d in range(DV):
                    v = rows_v[b, j * F + f, pl.ds(d * 16, 16)]
                    ss[d] = ss[d] + v
                    qq[d] = qq[d] + v * v
            s = c * SCH + j
            part = jnp.zeros((16,), jnp.float32)
            for d in range(DV):
                mean_v[s, pl.ds(d * 16, 16)] = ss[d] * inv
                part = part + (ss[d] * ss[d] - qq[d])
            fm2_v[c // 2, pl.ds((c % 2) * 64 + j * 16, 16)] = 0.5 * part

    start(0, 0)

    def pair(i, carry):
        c = i * 2
        start(c + 1, 1)
        wait(c, 0)
        reduce(c, 0)

        @pl.when(c + 2 < NCHUNK)
        def _():
            start(c + 2, 0)

        wait(c + 1, 1)
        reduce(c + 1, 1)
        return carry

    lax.fori_loop(0, NCHUNK // 2, pair, 0)
    pltpu.sync_copy(mean_v, mean_hbm.at[pl.ds(wid * BPW, BPW)])
    pltpu.sync_copy(fm2_v, fm2_hbm.at[pl.ds(wid * (BPW // 8), BPW // 8)])


_sc_reduce = functools.partial(
    pl.kernel,
    out_type=(jax.ShapeDtypeStruct((BSP, D), jnp.float32),
              jax.ShapeDtypeStruct((BSP // 8, D), jnp.float32)),
    mesh=plsc.VectorSubcoreMesh(core_axis_name="c", subcore_axis_name="s",
                                num_cores=NC, num_subcores=NS),
    scratch_types=[
        pltpu.VMEM((NCHUNK, RCH), jnp.int32),
        pltpu.VMEM((2, RCH, D), jnp.float32),
        pltpu.VMEM((BPW, D), jnp.float32),
        pltpu.VMEM((BPW // 8, D), jnp.float32),
        pltpu.SemaphoreType.DMA,
        pltpu.SemaphoreType.DMA,
    ],
)(_sc_body)


TB = 4096
NT = B // TB
NTH = BSP // TB


def _tc_raw_body(scal_ref, mean_ref, fm2_ref, si_ref, w1_ref, b1_ref, w2_ref,
                 fmr_ref, dpr_ref):
    m = mean_ref[...]
    h = lax.dot_general(m, w1_ref[...], (((1,), (1,)), ((), ())),
                        preferred_element_type=jnp.float32)
    h = jnp.maximum(h + b1_ref[...], 0.0)
    dp = lax.dot_general(h, w2_ref[...], (((1,), (1,)), ((), ())),
                         preferred_element_type=jnp.float32)
    fm1 = jnp.sum(si_ref[...].astype(jnp.float32), axis=1)
    fmr_ref[0, 0, :] = fm1 + jnp.sum(fm2_ref[...], axis=1)
    dpr_ref[0, 0, :] = dp[:, 0] + scal_ref[6]


def _tc_fin_body(scal_ref, fmr_ref, dpr_ref, out_ref):
    fmr = fmr_ref[...]
    dpr = dpr_ref[...]
    fmean = jnp.mean(fmr)
    fvar = jnp.mean((fmr - fmean) ** 2)
    dmean = jnp.mean(dpr)
    dvar = jnp.mean((dpr - dmean) ** 2)
    fm_n = scal_ref[0] * (fmr - fmean) * lax.rsqrt(fvar + EPS) + scal_ref[1]
    dp_n = scal_ref[2] * (dpr - dmean) * lax.rsqrt(dvar + EPS) + scal_ref[3]
    out_ref[...] = jax.nn.sigmoid(scal_ref[4] * fm_n + scal_ref[5] * dp_n)


def _tc_raw(scal, mean_h, fm2_h, si_h, w1, b1r, w2):
    return pl.pallas_call(
        _tc_raw_body,
        grid=(NTH,),
        in_specs=[
            pl.BlockSpec(memory_space=pltpu.SMEM),
            pl.BlockSpec((TB, D), lambda c: (c, 0)),
            pl.BlockSpec((TB, 16), lambda c: (c, 0)),
            pl.BlockSpec((TB, F), lambda c: (c, 0)),
            pl.BlockSpec((H, D), lambda c: (0, 0)),
            pl.BlockSpec((1, H), lambda c: (0, 0)),
            pl.BlockSpec((1, H), lambda c: (0, 0)),
        ],
        out_specs=[pl.BlockSpec((1, 1, TB), lambda c: (c, 0, 0)),
                   pl.BlockSpec((1, 1, TB), lambda c: (c, 0, 0))],
        out_shape=[jax.ShapeDtypeStruct((NTH, 1, TB), jnp.float32),
                   jax.ShapeDtypeStruct((NTH, 1, TB), jnp.float32)],
    )(scal, mean_h, fm2_h, si_h, w1, b1r, w2)


def kernel(sparse_inputs, emb_tables, w1, b1, w2, b2, fm_gamma, fm_beta,
           deep_gamma, deep_beta, combine_weight):
    si = sparse_inputs.astype(jnp.int32)
    flat = si + (jnp.arange(F, dtype=jnp.int32) * V)[None, :]
    idx = flat.reshape(NSPLIT, NW, NCHUNK, RCH)
    table = emb_tables.reshape(F * V, D)
    scal = jnp.concatenate([fm_gamma, fm_beta, deep_gamma, deep_beta,
                            combine_weight, b2]).astype(jnp.float32)
    b1r = b1.reshape(1, H)
    si_s = si.reshape(NSPLIT, BSP, F)

    means, fm2s = [], []
    for h in range(NSPLIT):
        mean_h, fm2_h = _sc_reduce(idx[h], table)
        means.append(mean_h)
        fm2s.append(fm2_h)

    fmrs, dprs = [], []
    for h in range(NSPLIT):
        fmr_h, dpr_h = _tc_raw(scal, means[h], fm2s[h].reshape(BSP, 16),
                               si_s[h], w1, b1r, w2)
        fmrs.append(fmr_h)
        dprs.append(dpr_h)

    fmr = jnp.concatenate(fmrs).reshape(NT, TB)
    dpr = jnp.concatenate(dprs).reshape(NT, TB)
    out = pl.pallas_call(
        _tc_fin_body,
        in_specs=[
            pl.BlockSpec(memory_space=pltpu.SMEM),
            pl.BlockSpec((NT, TB), lambda: (0, 0)),
            pl.BlockSpec((NT, TB), lambda: (0, 0)),
        ],
        out_specs=pl.BlockSpec((NT, TB), lambda: (0, 0)),
        out_shape=jax.ShapeDtypeStruct((NT, TB), jnp.float32),
    )(scal, fmr, dpr)
    return out.reshape(B, 1)
